# Initial kernel scaffold; baseline (speedup 1.0000x reference)
import jax
import jax.numpy as jnp
from jax.experimental import pallas as pl


def kernel(species_ids, type1_ids, type2_ids, base_stats, move_ids, move_type_ids, move_properties, item_ids, item_category_ids, ability_ids, effect_ids, W_species, W_ptype, W_stat1, b_stat1, W_stat2, b_stat2, W_pproj, b_pproj, W_move, W_mtype, W_prop1, b_prop1, W_prop2, b_prop2, W_mproj, b_mproj, W_item, W_icat, W_iproj, b_iproj, W_ability, W_effect, W_aproj, b_aproj):
    raise NotImplementedError("write your pallas kernel here")


# speedup vs baseline: 3.8096x; 3.8096x over previous
import functools

import jax
import jax.numpy as jnp
from jax import lax
from jax.experimental import pallas as pl


def _pad_rows(x, n):
    return jnp.pad(x, ((0, n - x.shape[0]),) + ((0, 0),) * (x.ndim - 1))


def _prep_kernel(wsp, wp1, wpt, wp2, wp3, wmv, wm1, wmt, wm2,
                 wit, wi1, wic, wi2, wab, wa1, wef, wa2,
                 psp, pt1, pt2, pmv, pmt, pit, pic, pab, pef):
    dot = functools.partial(jnp.dot, preferred_element_type=jnp.float32)
    psp[...] = dot(wsp[...], wp1[...])
    pt1[...] = dot(wpt[...], wp2[...])
    pt2[...] = dot(wpt[...], wp3[...])
    pmv[...] = dot(wmv[...], wm1[...])
    pmt[...] = dot(wmt[...], wm2[...])
    pit[...] = dot(wit[...], wi1[...])
    pic[...] = dot(wic[...], wi2[...])
    pab[...] = dot(wab[...], wa1[...])
    pef[...] = dot(wef[...], wa2[...])


def _main_kernel(sid, t1, t2, bs, mv, mt, mp, iid, ic, aid, ef,
                 psp, pt1, pt2, pmv, pmt, pit, pic, pab, pef,
                 ws1, bs1, ws2, bs2, wp4, bpp,
                 wq1, bq1, wq2, bq2, wm3, bmp, bip, bap,
                 pok_o, mov_o, itm_o, abl_o):
    f32 = jnp.float32
    dot = functools.partial(jnp.dot, preferred_element_type=f32)

    def onehot(ids, n):
        r = ids.shape[0]
        return (ids.reshape(r, 1) ==
                lax.broadcasted_iota(jnp.int32, (r, n), 1)).astype(f32)

    st = jnp.maximum(dot(bs[...], ws1[...]) + bs1[...], 0.0)
    st = dot(st, ws2[...]) + bs2[...]
    pok = dot(onehot(sid[0, 0], 2048), psp[...])
    pok += dot(onehot(t1[0, 0], 32), pt1[...])
    pok += dot(onehot(t2[0, 0], 32), pt2[...])
    pok += dot(st, wp4[...]) + bpp[...]
    pok_o[...] = pok.reshape(pok_o.shape)

    pe = jnp.maximum(dot(mp[...], wq1[...]) + bq1[...], 0.0)
    pe = dot(pe, wq2[...]) + bq2[...]
    mov = dot(onehot(mv[0, 0], 1024), pmv[...])
    mov += dot(onehot(mt[0, 0], 32), pmt[...])
    mov += dot(pe, wm3[...]) + bmp[...]
    mov_o[...] = mov.reshape(mov_o.shape)

    itm = dot(onehot(iid[0, 0], 512), pit[...])
    itm += dot(onehot(ic[0, 0], 32), pic[...]) + bip[...]
    itm_o[...] = itm.reshape(itm_o.shape)

    abl = dot(onehot(aid[0, 0], 512), pab[...])
    abl += dot(onehot(ef[0, 0], 32), pef[...]) + bap[...]
    abl_o[...] = abl.reshape(abl_o.shape)


def kernel(species_ids, type1_ids, type2_ids, base_stats, move_ids,
           move_type_ids, move_properties, item_ids, item_category_ids,
           ability_ids, effect_ids, W_species, W_ptype, W_stat1, b_stat1,
           W_stat2, b_stat2, W_pproj, b_pproj, W_move, W_mtype, W_prop1,
           b_prop1, W_prop2, b_prop2, W_mproj, b_mproj, W_item, W_icat,
           W_iproj, b_iproj, W_ability, W_effect, W_aproj, b_aproj):
    B, T = species_ids.shape
    M = move_ids.shape[2]
    N = B * T
    NM = N * M
    R = 256
    G = N // R
    RM = R * M
    pd, md, idm, ad = 128, 64, 32, 32

    f32 = jnp.float32

    vmem = functools.partial(pl.BlockSpec, memory_space=pl.ANY)
    prep_in = [
        _pad_rows(W_species, 2048), W_pproj[0:128],
        _pad_rows(W_ptype, 32), W_pproj[128:144], W_pproj[144:160],
        _pad_rows(W_move, 1024), W_mproj[0:64],
        _pad_rows(W_mtype, 32), W_mproj[64:80],
        _pad_rows(W_item, 512), W_iproj[0:32],
        _pad_rows(W_icat, 32), W_iproj[32:40],
        _pad_rows(W_ability, 512), W_aproj[0:32],
        _pad_rows(W_effect, 32), W_aproj[32:40],
    ]
    prep_out = [
        jax.ShapeDtypeStruct((2048, pd), f32),
        jax.ShapeDtypeStruct((32, pd), f32),
        jax.ShapeDtypeStruct((32, pd), f32),
        jax.ShapeDtypeStruct((1024, md), f32),
        jax.ShapeDtypeStruct((32, md), f32),
        jax.ShapeDtypeStruct((512, idm), f32),
        jax.ShapeDtypeStruct((32, idm), f32),
        jax.ShapeDtypeStruct((512, ad), f32),
        jax.ShapeDtypeStruct((32, ad), f32),
    ]
    tables = pl.pallas_call(
        _prep_kernel,
        out_shape=prep_out,
    )(*prep_in)

    sid = species_ids.reshape(G, 1, R).astype(jnp.int32)
    t1 = type1_ids.reshape(G, 1, R).astype(jnp.int32)
    t2 = type2_ids.reshape(G, 1, R).astype(jnp.int32)
    iid = item_ids.reshape(G, 1, R).astype(jnp.int32)
    ic = item_category_ids.reshape(G, 1, R).astype(jnp.int32)
    aid = ability_ids.reshape(G, 1, R).astype(jnp.int32)
    ef = effect_ids.reshape(G, 1, R).astype(jnp.int32)
    mv = move_ids.reshape(G, 1, RM).astype(jnp.int32)
    mt = move_type_ids.reshape(G, 1, RM).astype(jnp.int32)
    bs = jnp.pad(base_stats.reshape(N, 6), ((0, 0), (0, 2))).reshape(G, R, 8)
    mp = jnp.pad(move_properties.reshape(NM, 20),
                 ((0, 0), (0, 12))).reshape(G, RM, 32)

    ws1 = _pad_rows(W_stat1, 8)
    wq1 = _pad_rows(W_prop1, 32)
    row = lambda v: v.reshape(1, -1)

    idx_spec = lambda r: pl.BlockSpec((1, 1, r), lambda i: (i, 0, 0))
    dense_spec = lambda r, c: pl.BlockSpec((1, r, c), lambda i: (i, 0, 0))
    full = lambda *s: pl.BlockSpec(s, lambda i: (0,) * len(s))

    in_specs = (
        [idx_spec(R)] * 3
        + [pl.BlockSpec((1, R, 8), lambda i: (i, 0, 0))]
        + [idx_spec(RM)] * 2
        + [pl.BlockSpec((1, RM, 32), lambda i: (i, 0, 0))]
        + [idx_spec(R)] * 4
        + [full(*t.shape) for t in tables]
        + [full(8, 32), full(1, 32), full(32, 32), full(1, 32),
           full(32, pd), full(1, pd),
           full(32, 32), full(1, 32), full(32, 32), full(1, 32),
           full(32, md), full(1, md), full(1, idm), full(1, ad)]
    )
    out_specs = [
        dense_spec(R, pd), dense_spec(RM, md),
        dense_spec(R, idm), dense_spec(R, ad),
    ]
    out_shape = [
        jax.ShapeDtypeStruct((G, R, pd), f32),
        jax.ShapeDtypeStruct((G, RM, md), f32),
        jax.ShapeDtypeStruct((G, R, idm), f32),
        jax.ShapeDtypeStruct((G, R, ad), f32),
    ]
    pok, mov, itm, abl = pl.pallas_call(
        _main_kernel,
        grid=(G,),
        in_specs=in_specs,
        out_specs=out_specs,
        out_shape=out_shape,
    )(sid, t1, t2, bs.reshape(G, R, 8), mv, mt, mp, iid, ic, aid, ef,
      *tables,
      ws1, row(b_stat1), W_stat2, row(b_stat2), W_pproj[160:192],
      row(b_pproj), wq1, row(b_prop1), W_prop2, row(b_prop2),
      W_mproj[80:112], row(b_mproj), row(b_iproj), row(b_aproj))

    return (pok.reshape(B, T, pd), mov.reshape(B, T, M, md),
            itm.reshape(B, T, idm), abl.reshape(B, T, ad))
